# Initial kernel scaffold; baseline (speedup 1.0000x reference)
import jax
import jax.numpy as jnp
from jax.experimental import pallas as pl


def kernel(idx, targets, table):
    raise NotImplementedError("write your pallas kernel here")


# speedup vs baseline: 1.7338x; 1.7338x over previous
import functools

import jax
import jax.numpy as jnp
from jax import lax
from jax.experimental import pallas as pl
from jax.experimental.pallas import tpu as pltpu
from jax.experimental.pallas import tpu_sc as plsc

NC = 2
NS = 16
LANES = 16
NW = NC * NS


def _lse_body(t_ref, o_ref):
    x = t_ref[...]
    m = jnp.max(x, axis=1, keepdims=True)
    s = jnp.sum(jnp.exp(x - m), axis=1, keepdims=True)
    o_ref[...] = m + jnp.log(s)


def _loss_body(p_ref, o_ref, *, inv_n):
    o_ref[...] = jnp.full((1, 1), inv_n, jnp.float32) * jnp.sum(p_ref[...])


def _sc_gather_fn(V, C, N, CH):
    b_per_w = N // NW
    n_ch = b_per_w // CH
    mesh = plsc.VectorSubcoreMesh(core_axis_name="c", subcore_axis_name="s")

    def body(table_h, idx_h, tgt_h, lse_h, out_h, part_h,
             idx_v, tgt_v, lse_v, rows_v, acc_v, gsem0, gsem1, osem0, osem1):
        cid = lax.axis_index("c")
        sid = lax.axis_index("s")
        wid = sid * NC + cid
        base = wid * b_per_w
        pltpu.sync_copy(idx_h.at[pl.ds(base, b_per_w)], idx_v)
        pltpu.sync_copy(tgt_h.at[pl.ds(base, b_per_w)], tgt_v)
        pltpu.sync_copy(lse_h, lse_v)

        gsems = (gsem0, gsem1)
        osems = (osem0, osem1)
        iot = lax.iota(jnp.int32, LANES)
        gcp = [None] * n_ch
        ocp = [None] * n_ch

        def start_gather(c):
            p = c % 2
            gcp[c] = pltpu.async_copy(
                table_h.at[idx_v.at[pl.ds(c * CH, CH)]], rows_v.at[p], gsems[p])

        start_gather(0)
        acc = jnp.zeros((LANES,), jnp.float32)
        for c in range(n_ch):
            p = c % 2
            gcp[c].wait()
            if c + 1 < n_ch:
                if c >= 1:
                    ocp[c - 1].wait()
                start_gather(c + 1)
            for g in range(CH // LANES):
                off = c * CH + g * LANES
                idx_vals = idx_v[pl.ds(off, LANES)]
                tgt_vals = tgt_v[pl.ds(off, LANES)]
                lse_g = plsc.load_gather(lse_v, [idx_vals])
                t_log = plsc.load_gather(rows_v.at[p], [iot + g * LANES, tgt_vals])
                acc = acc + (lse_g - t_log)
            ocp[c] = pltpu.async_copy(
                rows_v.at[p], out_h.at[pl.ds(base + c * CH, CH)], osems[p])
        ocp[n_ch - 2].wait()
        ocp[n_ch - 1].wait()
        acc_v[...] = acc
        pltpu.sync_copy(acc_v, part_h.at[pl.ds(wid * LANES, LANES)])

    return pl.kernel(
        body,
        out_type=[
            jax.ShapeDtypeStruct((N, C), jnp.float32),
            jax.ShapeDtypeStruct((NW * LANES,), jnp.float32),
        ],
        mesh=mesh,
        compiler_params=pltpu.CompilerParams(
            use_tc_tiling_on_sc=False, needs_layout_passes=False),
        scratch_types=[
            pltpu.VMEM((b_per_w,), jnp.int32),
            pltpu.VMEM((b_per_w,), jnp.int32),
            pltpu.VMEM((V,), jnp.float32),
            pltpu.VMEM((2, CH, C), jnp.float32),
            pltpu.VMEM((LANES,), jnp.float32),
            pltpu.SemaphoreType.DMA,
            pltpu.SemaphoreType.DMA,
            pltpu.SemaphoreType.DMA,
            pltpu.SemaphoreType.DMA,
        ],
    )


def kernel(idx, targets, table):
    V, C = table.shape
    Bb, Tt = idx.shape
    N = Bb * Tt
    idx_f = idx.reshape(N).astype(jnp.int32)
    tgt_f = targets.reshape(N).astype(jnp.int32)

    lse = pl.pallas_call(
        _lse_body,
        out_shape=jax.ShapeDtypeStruct((V, 1), jnp.float32),
    )(table).reshape(V)

    out, part = _sc_gather_fn(V, C, N, CH=32)(table, idx_f, tgt_f, lse)

    loss = pl.pallas_call(
        functools.partial(_loss_body, inv_n=1.0 / N),
        out_shape=jax.ShapeDtypeStruct((1, 1), jnp.float32),
    )(part.reshape(NW, LANES))[0, 0]

    return out, loss


# speedup vs baseline: 2.7430x; 1.5820x over previous
import functools

import jax
import jax.numpy as jnp
from jax import lax
from jax.experimental import pallas as pl
from jax.experimental.pallas import tpu as pltpu
from jax.experimental.pallas import tpu_sc as plsc

NC = 2
NS = 16
LANES = 16
NW = NC * NS


def _lse_pad_body(t_ref, lse_ref, tp_ref):
    x = t_ref[...]
    m = jnp.max(x, axis=1, keepdims=True)
    s = jnp.sum(jnp.exp(x - m), axis=1, keepdims=True)
    lse_ref[...] = m + jnp.log(s)
    tp_ref[...] = jnp.pad(x, ((0, 0), (0, tp_ref.shape[1] - x.shape[1])))


def _loss_body(p_ref, o_ref, *, inv_n):
    o_ref[...] = jnp.full((1, 1), inv_n, jnp.float32) * jnp.sum(p_ref[...])


def _sc_gather_fn(V, C, CP, N, CH):
    b_per_w = N // NW
    n_ch = b_per_w // CH
    mesh = plsc.VectorSubcoreMesh(core_axis_name="c", subcore_axis_name="s")

    def body(table_h, idx_h, tgt_h, lse_h, out_h, part_h,
             idx_v, tgt_v, lse_v, rows_v, acc_v, gsem0, gsem1, osem0, osem1):
        cid = lax.axis_index("c")
        sid = lax.axis_index("s")
        wid = sid * NC + cid
        base = wid * b_per_w
        pltpu.sync_copy(idx_h.at[pl.ds(base, b_per_w)], idx_v)
        pltpu.sync_copy(tgt_h.at[pl.ds(base, b_per_w)], tgt_v)
        pltpu.sync_copy(lse_h, lse_v)

        gsems = (gsem0, gsem1)
        osems = (osem0, osem1)
        iot = lax.iota(jnp.int32, LANES)
        gcp = [None] * n_ch
        ocp = [None] * n_ch

        def start_gather(c):
            p = c % 2
            gcp[c] = pltpu.async_copy(
                table_h.at[idx_v.at[pl.ds(c * CH, CH)]], rows_v.at[p], gsems[p])

        start_gather(0)
        acc = jnp.zeros((LANES,), jnp.float32)
        for c in range(n_ch):
            p = c % 2
            gcp[c].wait()
            if c + 1 < n_ch:
                if c >= 1:
                    ocp[c - 1].wait()
                start_gather(c + 1)
            for g in range(CH // LANES):
                off = c * CH + g * LANES
                idx_vals = idx_v[pl.ds(off, LANES)]
                tgt_vals = tgt_v[pl.ds(off, LANES)]
                lse_g = plsc.load_gather(lse_v, [idx_vals])
                t_log = plsc.load_gather(rows_v.at[p], [iot + g * LANES, tgt_vals])
                acc = acc + (lse_g - t_log)
            ocp[c] = pltpu.async_copy(
                rows_v.at[p], out_h.at[pl.ds(base + c * CH, CH)], osems[p])
        ocp[n_ch - 2].wait()
        ocp[n_ch - 1].wait()
        acc_v[...] = acc
        pltpu.sync_copy(acc_v, part_h.at[pl.ds(wid * LANES, LANES)])

    return pl.kernel(
        body,
        out_type=[
            jax.ShapeDtypeStruct((N, CP), jnp.float32),
            jax.ShapeDtypeStruct((NW * LANES,), jnp.float32),
        ],
        mesh=mesh,
        compiler_params=pltpu.CompilerParams(needs_layout_passes=False),
        scratch_types=[
            pltpu.VMEM((b_per_w,), jnp.int32),
            pltpu.VMEM((b_per_w,), jnp.int32),
            pltpu.VMEM((V,), jnp.float32),
            pltpu.VMEM((2, CH, CP), jnp.float32),
            pltpu.VMEM((LANES,), jnp.float32),
            pltpu.SemaphoreType.DMA,
            pltpu.SemaphoreType.DMA,
            pltpu.SemaphoreType.DMA,
            pltpu.SemaphoreType.DMA,
        ],
    )


def kernel(idx, targets, table):
    V, C = table.shape
    CP = (C + 127) // 128 * 128
    Bb, Tt = idx.shape
    N = Bb * Tt
    idx_f = idx.reshape(N).astype(jnp.int32)
    tgt_f = targets.reshape(N).astype(jnp.int32)

    lse, table_p = pl.pallas_call(
        _lse_pad_body,
        out_shape=[
            jax.ShapeDtypeStruct((V, 1), jnp.float32),
            jax.ShapeDtypeStruct((V, CP), jnp.float32),
        ],
    )(table)

    out_p, part = _sc_gather_fn(V, C, CP, N, CH=32)(
        table_p, idx_f, tgt_f, lse.reshape(V))
    out = out_p[:, :C]

    loss = pl.pallas_call(
        functools.partial(_loss_body, inv_n=1.0 / N),
        out_shape=jax.ShapeDtypeStruct((1, 1), jnp.float32),
    )(part.reshape(NW, LANES))[0, 0]

    return out, loss
